# Initial kernel scaffold; baseline (speedup 1.0000x reference)
import jax
import jax.numpy as jnp
from jax.experimental import pallas as pl


def kernel(x, code_book):
    raise NotImplementedError("write your pallas kernel here")


# speedup vs baseline: 1.8522x; 1.8522x over previous
import functools

import jax
import jax.numpy as jnp
from jax import lax
from jax.experimental import pallas as pl
from jax.experimental.pallas import tpu as pltpu

NCB, NPOINT, NDIM = 8, 4096, 64
CB = 1024
P = 512
NPB = NPOINT // P


def _vq_body(x_ref, cbt_ref, dist_ref, oh_ref, xhat_ref, idx_ref):
    g = pl.program_id(0)
    x = x_ref[0]
    cbt = cbt_ref[0]
    xn = jnp.sum(x * x, axis=1, keepdims=True)
    cn = jnp.sum(cbt * cbt, axis=0, keepdims=True)
    prod = jax.lax.dot_general(x, cbt, (((1,), (0,)), ((), ())),
                               preferred_element_type=jnp.float32)
    dist = (xn + cn - 2.0 * prod) * (1.0 / NDIM)

    iota = lax.broadcasted_iota(jnp.int32, (P, CB), 1)
    m = jnp.min(dist, axis=1, keepdims=True)
    cand = jnp.where(dist == m, iota, CB)
    idx = jnp.min(cand, axis=1, keepdims=True)
    one_hot = (iota == idx).astype(jnp.float32)

    dist_ref[0] = dist
    oh_ref[0] = one_hot
    xhat_ref[0] = jax.lax.dot_general(
        one_hot, cbt, (((1,), (1,)), ((), ())),
        preferred_element_type=jnp.float32)
    idx_ref[0] = idx + g * CB


@functools.partial(jax.jit)
def _vq_tc(x, cb_t):
    grid = (NCB, NPB)
    return pl.pallas_call(
        _vq_body,
        grid=grid,
        in_specs=[
            pl.BlockSpec((1, P, NDIM), lambda g, p: (g, p, 0)),
            pl.BlockSpec((1, NDIM, CB), lambda g, p: (g, 0, 0)),
        ],
        out_specs=[
            pl.BlockSpec((1, P, CB), lambda g, p: (g, p, 0)),
            pl.BlockSpec((1, P, CB), lambda g, p: (g, p, 0)),
            pl.BlockSpec((1, P, NDIM), lambda g, p: (g, p, 0)),
            pl.BlockSpec((1, P, 1), lambda g, p: (g, p, 0)),
        ],
        out_shape=[
            jax.ShapeDtypeStruct((NCB, NPOINT, CB), jnp.float32),
            jax.ShapeDtypeStruct((NCB, NPOINT, CB), jnp.float32),
            jax.ShapeDtypeStruct((NCB, NPOINT, NDIM), jnp.float32),
            jax.ShapeDtypeStruct((NCB, NPOINT, 1), jnp.int32),
        ],
        compiler_params=pltpu.CompilerParams(
            dimension_semantics=("parallel", "arbitrary")),
    )(x, cb_t)


def kernel(x, code_book):
    cb_t = jnp.transpose(code_book, (0, 2, 1))
    dist, one_hot, x_hat, _idx = _vq_tc(x, cb_t)
    return (x_hat, one_hot, dist)
